# baseline (speedup 1.0000x reference)
import functools

import jax
import jax.numpy as jnp
from jax import lax
from jax.experimental import pallas as pl
from jax.experimental.pallas import tpu as pltpu
from jax.experimental.pallas import tpu_sc as plsc

INPUT_SEQ = 6
STEPS = 2
D = 3
RADIUS = 0.015
MP = 10
LAT = 128

_NC = 2
_NS = 16
_NW = _NC * _NS
_CH = 128


def _ln(x, s, b):
    m = jnp.mean(x, axis=-1, keepdims=True)
    xc = x - m
    v = jnp.mean(xc * xc, axis=-1, keepdims=True)
    return xc * lax.rsqrt(v + 1e-5) * s + b


def _node_encoder_body(cur_ref, pt_ref, temb_ref, w1_ref, b1_ref, w2_ref,
                       b2_ref, ls_ref, lb_ref, ws_ref, wd_ref,
                       h_ref, ps_ref, pd_ref, r8_ref):
    cur = cur_ref[...]
    vel = cur[:, 3:18] - cur[:, 0:15]
    recent = cur[:, 15:18]
    bdist = jnp.clip(jnp.minimum(recent, 1.0 - recent) / RADIUS, -1.0, 1.0)
    w1 = w1_ref[...]
    pt = pt_ref[...]
    onehot = (pt == lax.broadcasted_iota(jnp.int32, (pt.shape[0], 9), 1)
              ).astype(jnp.float32)
    emb_proj = temb_ref[...] @ w1[21:37]
    ctx = (0.5 * w1[37] + 0.00025 * w1[38] + 0.2 * w1[39])[None, :]
    pre = (vel @ w1[0:15] + recent @ w1[15:18] + bdist @ w1[18:21]
           + onehot @ emb_proj + ctx + b1_ref[...])
    hid = jnp.maximum(pre, 0.0)
    h = _ln(hid @ w2_ref[...] + b2_ref[...], ls_ref[...], lb_ref[...])
    h_ref[...] = h
    ps_ref[...] = h @ ws_ref[...]
    pd_ref[...] = h @ wd_ref[...]
    r8_ref[...] = jnp.concatenate(
        [recent, jnp.zeros((cur.shape[0], LAT - 3), jnp.float32)], axis=1)


def _node_encoder(cur18, ptype2, type_emb, w1, b1, w2, b2, ls, lb, ws, wd):
    n = cur18.shape[0]
    bn = 2000
    grid = (n // bn,)
    blk_n = lambda c: pl.BlockSpec((bn, c), lambda i: (i, 0))
    full = lambda a: pl.BlockSpec(a.shape, lambda i: (0,) * a.ndim)
    return pl.pallas_call(
        _node_encoder_body,
        grid=grid,
        in_specs=[blk_n(18), blk_n(1), full(type_emb), full(w1), full(b1),
                  full(w2), full(b2), full(ls), full(lb), full(ws), full(wd)],
        out_specs=[blk_n(LAT), blk_n(LAT), blk_n(LAT), blk_n(LAT)],
        out_shape=[jax.ShapeDtypeStruct((n, LAT), jnp.float32)] * 4,
    )(cur18, ptype2, type_emb, w1, b1, w2, b2, ls, lb, ws, wd)


def _edge_encoder_body(rs_ref, rd_ref, w1_ref, b1_ref, w2_ref, b2_ref,
                       ls_ref, lb_ref, e_ref):
    disp = (rs_ref[...][:, 0:3] - rd_ref[...][:, 0:3]) * (1.0 / RADIUS)
    nrm = jnp.sqrt(jnp.sum(disp * disp, axis=1, keepdims=True))
    w1 = w1_ref[...]
    pre = disp @ w1[0:3] + nrm @ w1[3:4] + b1_ref[...]
    hid = jnp.maximum(pre, 0.0)
    e_ref[...] = _ln(hid @ w2_ref[...] + b2_ref[...], ls_ref[...],
                     lb_ref[...])


def _edge_encoder(rs8, rd8, w1, b1, w2, b2, ls, lb):
    e = rs8.shape[0]
    be = 4096
    grid = (e // be,)
    blk = lambda c: pl.BlockSpec((be, c), lambda i: (i, 0))
    full = lambda a: pl.BlockSpec(a.shape, lambda i: (0,) * a.ndim)
    return pl.pallas_call(
        _edge_encoder_body,
        grid=grid,
        in_specs=[blk(LAT), blk(LAT), full(w1), full(b1), full(w2), full(b2),
                  full(ls), full(lb)],
        out_specs=blk(LAT),
        out_shape=jax.ShapeDtypeStruct((e, LAT), jnp.float32),
    )(rs8, rd8, w1, b1, w2, b2, ls, lb)


def _edge_mp_body(e_ref, gps_ref, gpd_ref, we_ref, b1_ref, w2_ref, b2_ref,
                  out_ref):
    ev = e_ref[...]
    t = jnp.maximum(ev @ we_ref[...] + gps_ref[...] + gpd_ref[...]
                    + b1_ref[...], 0.0)
    out_ref[...] = ev + t @ w2_ref[...] + b2_ref[...]


def _edge_mp(e, gps, gpd, we, b1, w2, b2):
    ne = e.shape[0]
    be = 4096
    grid = (ne // be,)
    blk = pl.BlockSpec((be, LAT), lambda i: (i, 0))
    full = lambda a: pl.BlockSpec(a.shape, lambda i: (0,) * a.ndim)
    return pl.pallas_call(
        _edge_mp_body,
        grid=grid,
        in_specs=[blk, blk, blk, full(we), full(b1), full(w2), full(b2)],
        out_specs=blk,
        out_shape=jax.ShapeDtypeStruct((ne, LAT), jnp.float32),
    )(e, gps, gpd, we, b1, w2, b2)


def _node_mp_body(h_ref, agg_ref, aggb_ref, wh_ref, wa_ref, b1_ref, w2_ref,
                  b2_ref, ws_ref, wd_ref, hn_ref, ps_ref, pd_ref):
    h = h_ref[...]
    agg = (agg_ref[0] + agg_ref[1]) + (aggb_ref[0] + aggb_ref[1])
    t = jnp.maximum(h @ wh_ref[...] + agg @ wa_ref[...] + b1_ref[...], 0.0)
    hn = h + t @ w2_ref[...] + b2_ref[...]
    hn_ref[...] = hn
    ps_ref[...] = hn @ ws_ref[...]
    pd_ref[...] = hn @ wd_ref[...]


def _node_mp(h, agg2, agg2b, wh, wa, b1, w2, b2, ws, wd):
    n = h.shape[0]
    bn = 2000
    grid = (n // bn,)
    blk = pl.BlockSpec((bn, LAT), lambda i: (i, 0))
    blk2 = pl.BlockSpec((2, bn, LAT), lambda i: (0, i, 0))
    full = lambda a: pl.BlockSpec(a.shape, lambda i: (0,) * a.ndim)
    return pl.pallas_call(
        _node_mp_body,
        grid=grid,
        in_specs=[blk, blk2, blk2, full(wh), full(wa), full(b1), full(w2),
                  full(b2), full(ws), full(wd)],
        out_specs=[blk, blk, blk],
        out_shape=[jax.ShapeDtypeStruct((n, LAT), jnp.float32)] * 3,
    )(h, agg2, agg2b, wh, wa, b1, w2, b2, ws, wd)


def _node_last_body(h_ref, agg_ref, aggb_ref, wh_ref, wa_ref, b1_ref,
                    w2_ref, b2_ref, dw1_ref, db1_ref, dw2_ref, db2_ref,
                    cur_ref, gt_ref, kin_ref, nxt_ref):
    h = h_ref[...]
    agg = (agg_ref[0] + agg_ref[1]) + (aggb_ref[0] + aggb_ref[1])
    t = jnp.maximum(h @ wh_ref[...] + agg @ wa_ref[...] + b1_ref[...], 0.0)
    hn = h + t @ w2_ref[...] + b2_ref[...]
    dh = jnp.maximum(hn @ dw1_ref[...] + db1_ref[...], 0.0)
    acc = (dh @ dw2_ref[...] + db2_ref[...]) * 1e-3
    cur = cur_ref[...]
    recent = cur[:, 15:18]
    vlast = recent - cur[:, 12:15]
    nxt = recent + vlast + acc[:, 0:3]
    kin = kin_ref[...] != 0
    nxt_ref[...] = jnp.where(kin, gt_ref[...], nxt)


def _node_last(h, agg2, agg2b, wh, wa, b1, w2, b2, dw1, db1, dw2p, db2p,
               cur18, gt_step, kin2):
    n = h.shape[0]
    bn = 2000
    grid = (n // bn,)
    blk = lambda c: pl.BlockSpec((bn, c), lambda i: (i, 0))
    blk2 = pl.BlockSpec((2, bn, LAT), lambda i: (0, i, 0))
    full = lambda a: pl.BlockSpec(a.shape, lambda i: (0,) * a.ndim)
    return pl.pallas_call(
        _node_last_body,
        grid=grid,
        in_specs=[blk(LAT), blk2, blk2, full(wh), full(wa), full(b1),
                  full(w2), full(b2), full(dw1), full(db1), full(dw2p),
                  full(db2p), blk(18), blk(D), blk(1)],
        out_specs=blk(D),
        out_shape=jax.ShapeDtypeStruct((n, D), jnp.float32),
    )(h, agg2, agg2b, wh, wa, b1, w2, b2, dw1, db1, dw2p, db2p, cur18,
      gt_step, kin2)


def _loss_body(p_ref, g_ref, nk_ref, loss_ref):
    d = p_ref[...] - g_ref[...]
    sq = jnp.sum(d * d, axis=2)
    nk = nk_ref[...]
    num = jnp.sum(sq * nk)
    loss_ref[...] = (num / jnp.sum(nk)).reshape(1, 1)


def _loss(preds, gt_p, nonkin):
    n = nonkin.shape[1]
    full = lambda a: pl.BlockSpec(a.shape, lambda: (0,) * a.ndim)
    out = pl.pallas_call(
        _loss_body,
        in_specs=[full(preds), full(gt_p), full(nonkin)],
        out_specs=pl.BlockSpec((1, 1), lambda: (0, 0)),
        out_shape=jax.ShapeDtypeStruct((1, 1), jnp.float32),
    )(preds, gt_p, nonkin)
    return out[0, 0]


def _sc_gather2(ps, pd, src2d, dst2d):
    n_chunks, ch = src2d.shape
    n_edges = n_chunks * ch
    w = ps.shape[1]
    dt = ps.dtype
    npt = n_chunks // _NS
    nb = 4
    mesh = plsc.VectorSubcoreMesh(core_axis_name="c", subcore_axis_name="s")

    @functools.partial(
        pl.kernel,
        out_type=[jax.ShapeDtypeStruct((n_edges, w), dt)] * 2,
        mesh=mesh,
        scratch_types=[
            pltpu.VMEM((npt, _CH), jnp.int32),
            pltpu.VMEM((nb, _CH, w), dt),
            pltpu.SemaphoreType.DMA((nb,)),
            pltpu.SemaphoreType.DMA((nb,)),
        ],
    )
    def k(ps_hbm, pd_hbm, src_hbm, dst_hbm, gps_hbm, gpd_hbm,
          idx, bufs, semg, semw):
        cid = lax.axis_index("c")
        row0 = lax.axis_index("s") * npt

        def pipeline(tab_hbm, idx2d_hbm, out_hbm):
            pltpu.sync_copy(idx2d_hbm.at[pl.ds(row0, npt)], idx)
            for j in range(nb - 1):
                pltpu.async_copy(tab_hbm.at[idx.at[j]], bufs.at[j],
                                 semg.at[j])

            def body(g, _):
                slot = lax.rem(g, nb)
                pltpu.make_async_copy(tab_hbm.at[idx.at[g]], bufs.at[slot],
                                      semg.at[slot]).wait()
                i = g + nb - 1

                @pl.when(i < npt)
                def _():
                    islot = lax.rem(i, nb)

                    @pl.when(i >= nb)
                    def _():
                        off2 = (row0 + i - nb) * _CH
                        pltpu.make_async_copy(
                            bufs.at[islot], out_hbm.at[pl.ds(off2, _CH)],
                            semw.at[islot]).wait()

                    pltpu.async_copy(tab_hbm.at[idx.at[i]], bufs.at[islot],
                                     semg.at[islot])

                off = (row0 + g) * _CH
                pltpu.async_copy(bufs.at[slot], out_hbm.at[pl.ds(off, _CH)],
                                 semw.at[slot])
                return ()

            lax.fori_loop(0, npt, body, ())
            for j in range(nb):
                g = npt - nb + j
                off = (row0 + g) * _CH
                pltpu.make_async_copy(bufs.at[g % nb],
                                      out_hbm.at[pl.ds(off, _CH)],
                                      semw.at[g % nb]).wait()

        @pl.when(cid == 0)
        def _():
            pipeline(ps_hbm, src_hbm, gps_hbm)

        @pl.when(cid == 1)
        def _():
            pipeline(pd_hbm, dst_hbm, gpd_hbm)

    return k(ps, pd, src2d, dst2d)


def _sc_scatter(e, dst2d, zeros_n):
    n_edges, lat = e.shape
    ch = dst2d.shape[1]
    n = zeros_n.shape[0]
    rows_t = n // _NS
    n_chunks_s = (n_edges // ch) // _NW
    mesh = plsc.VectorSubcoreMesh(core_axis_name="c", subcore_axis_name="s")

    @functools.partial(
        pl.kernel,
        out_type=jax.ShapeDtypeStruct((_NC, n, lat), e.dtype),
        mesh=mesh,
        scratch_types=[
            pltpu.VMEM((n_chunks_s, ch), jnp.int32),
            pltpu.VMEM((2, ch, lat), e.dtype),
            pltpu.VMEM_SHARED((n, lat), e.dtype),
            pltpu.SemaphoreType.DMA,
            pltpu.SemaphoreType.DMA,
        ],
    )
    def k(e_hbm, dst_hbm, z_hbm, out_hbm, idxd, buf, agg, seml, sems):
        cid = lax.axis_index("c")
        sid = lax.axis_index("s")
        row0 = (cid * _NS + sid) * n_chunks_s
        pltpu.sync_copy(dst_hbm.at[pl.ds(row0, n_chunks_s)], idxd)
        r0 = sid * rows_t
        pltpu.sync_copy(z_hbm.at[pl.ds(r0, rows_t)],
                        agg.at[pl.ds(r0, rows_t)])
        plsc.subcore_barrier()
        pltpu.async_copy(e_hbm.at[pl.ds(row0 * ch, ch)], buf.at[0], seml)

        def body(g, _):
            cur_s = lax.rem(g, 2)
            nxt_s = 1 - cur_s
            pltpu.make_async_copy(e_hbm.at[pl.ds((row0 + g) * ch, ch)],
                                  buf.at[cur_s], seml).wait()

            @pl.when(g >= 1)
            def _():
                pltpu.make_async_copy(buf.at[nxt_s], agg.at[idxd.at[g - 1]],
                                      sems).wait()

            @pl.when(g + 1 < n_chunks_s)
            def _():
                pltpu.async_copy(
                    e_hbm.at[pl.ds((row0 + g + 1) * ch, ch)],
                    buf.at[nxt_s], seml)

            pltpu.async_copy(buf.at[cur_s], agg.at[idxd.at[g]], sems,
                             add=True)
            return ()

        lax.fori_loop(0, n_chunks_s, body, ())
        last = n_chunks_s - 1
        pltpu.make_async_copy(buf.at[last % 2], agg.at[idxd.at[last]],
                              sems).wait()
        plsc.subcore_barrier()
        pltpu.sync_copy(agg.at[pl.ds(r0, rows_t)],
                        out_hbm.at[cid, pl.ds(r0, rows_t)])

    return k(e, dst2d, zeros_n)


def kernel(position, step_context, type_emb, enW1, enb1, enW2, enb2, enls,
           enlb, eeW1, eeb1, eeW2, eeb2, eels, eelb, mpeW1, mpeb1, mpeW2,
           mpeb2, mpnW1, mpnb1, mpnW2, mpnb2, decW1, decb1, decW2, decb2,
           edge_index, particle_type):
    n = position.shape[0]
    n_edges = edge_index.shape[1]
    quantum = _NW * _CH
    e_pad = -(-n_edges // quantum) * quantum
    pad = e_pad - n_edges
    src_p = jnp.concatenate([edge_index[0], jnp.zeros((pad,), jnp.int32)])
    dst_g = jnp.concatenate([edge_index[1], jnp.zeros((pad,), jnp.int32)])
    dst_s = jnp.concatenate([edge_index[1], jnp.full((pad,), n, jnp.int32)])
    src2d = src_p.reshape(e_pad // _CH, _CH)
    dst2d_g = dst_g.reshape(e_pad // _CH, _CH)
    dst2d_s = dst_s.reshape(e_pad // 64, 64)
    rows_t = -(-(n + 1) // (_NS * 16)) * 16
    n_pad = rows_t * _NS
    kin = particle_type == 3
    kin2 = kin.astype(jnp.int32)[:, None]
    ptype2 = particle_type[:, None]
    initial = position[:, :INPUT_SEQ]
    gt = position[:, INPUT_SEQ:INPUT_SEQ + STEPS]
    zeros_n = jnp.zeros((n_pad, LAT), jnp.float32)

    we = [mpeW1[m][0:LAT] for m in range(MP)]
    ws = [mpeW1[m][LAT:2 * LAT] for m in range(MP)]
    wd = [mpeW1[m][2 * LAT:3 * LAT] for m in range(MP)]
    wh = [mpnW1[m][0:LAT] for m in range(MP)]
    wa = [mpnW1[m][LAT:2 * LAT] for m in range(MP)]
    row = lambda v: v.reshape(1, LAT)
    meb1 = [row(mpeb1[m]) for m in range(MP)]
    meb2 = [row(mpeb2[m]) for m in range(MP)]
    mew2 = [mpeW2[m] for m in range(MP)]
    mnb1 = [row(mpnb1[m]) for m in range(MP)]
    mnb2 = [row(mpnb2[m]) for m in range(MP)]
    dw2p = jnp.zeros((LAT, 8), jnp.float32).at[:, 0:D].set(decW2)
    db2p = jnp.zeros((1, 8), jnp.float32).at[0, 0:D].set(decb2)

    hc = (e_pad // _CH) // 2
    hcs = (e_pad // 64) // 2
    srcH = [src2d[:hc], src2d[hc:]]
    dstgH = [dst2d_g[:hc], dst2d_g[hc:]]
    dstsH = [dst2d_s[:hcs], dst2d_s[hcs:]]

    cur = initial
    preds = []
    for step in range(STEPS):
        cur18 = cur.reshape(n, INPUT_SEQ * D)
        h, psn, pdn, r8 = _node_encoder(cur18, ptype2, type_emb, enW1,
                                        row(enb1), enW2, row(enb2),
                                        row(enls), row(enlb), ws[0], wd[0])
        eh = []
        for hx in range(2):
            rs8, rd8 = _sc_gather2(r8, r8, srcH[hx], dstgH[hx])
            eh.append(_edge_encoder(rs8, rd8, eeW1, row(eeb1), eeW2,
                                    row(eeb2), row(eels), row(eelb)))
        for m in range(MP):
            aggh = []
            for hx in range(2):
                gps, gpd = _sc_gather2(psn, pdn, srcH[hx], dstgH[hx])
                eh[hx] = _edge_mp(eh[hx], gps, gpd, we[m], meb1[m], mew2[m],
                                  meb2[m])
                aggh.append(_sc_scatter(eh[hx], dstsH[hx], zeros_n))
            if m + 1 < MP:
                h, psn, pdn = _node_mp(h, aggh[0], aggh[1], wh[m], wa[m],
                                       mnb1[m], mpnW2[m], mnb2[m], ws[m + 1],
                                       wd[m + 1])
            else:
                nxt = _node_last(h, aggh[0], aggh[1], wh[m], wa[m], mnb1[m],
                                 mpnW2[m], mnb2[m], decW1, row(decb1), dw2p,
                                 db2p, cur18, gt[:, step], kin2)
        preds.append(nxt)
        cur = jnp.concatenate([cur[:, 1:], nxt[:, None, :]], axis=1)

    predictions = jnp.stack(preds)
    gt_p = jnp.transpose(gt, (1, 0, 2))
    nonkin = (~kin).astype(jnp.float32)[None, :]
    loss = _loss(predictions, gt_p, nonkin)
    return (loss, predictions, gt_p)


# speedup vs baseline: 1.0552x; 1.0552x over previous
import functools

import jax
import jax.numpy as jnp
from jax import lax
from jax.experimental import pallas as pl
from jax.experimental.pallas import tpu as pltpu
from jax.experimental.pallas import tpu_sc as plsc

INPUT_SEQ = 6
STEPS = 2
D = 3
RADIUS = 0.015
MP = 10
LAT = 128

_NC = 2
_NS = 16
_NW = _NC * _NS
_CH = 128


def _ln(x, s, b):
    m = jnp.mean(x, axis=-1, keepdims=True)
    xc = x - m
    v = jnp.mean(xc * xc, axis=-1, keepdims=True)
    return xc * lax.rsqrt(v + 1e-5) * s + b


def _node_encoder_body(cur_ref, pt_ref, temb_ref, w1_ref, b1_ref, w2_ref,
                       b2_ref, ls_ref, lb_ref, ws_ref, wd_ref,
                       h_ref, ps_ref, pd_ref, r8_ref):
    cur = cur_ref[...]
    vel = cur[:, 3:18] - cur[:, 0:15]
    recent = cur[:, 15:18]
    bdist = jnp.clip(jnp.minimum(recent, 1.0 - recent) / RADIUS, -1.0, 1.0)
    w1 = w1_ref[...]
    pt = pt_ref[...]
    onehot = (pt == lax.broadcasted_iota(jnp.int32, (pt.shape[0], 9), 1)
              ).astype(jnp.float32)
    emb_proj = temb_ref[...] @ w1[21:37]
    ctx = (0.5 * w1[37] + 0.00025 * w1[38] + 0.2 * w1[39])[None, :]
    pre = (vel @ w1[0:15] + recent @ w1[15:18] + bdist @ w1[18:21]
           + onehot @ emb_proj + ctx + b1_ref[...])
    hid = jnp.maximum(pre, 0.0)
    h = _ln(hid @ w2_ref[...] + b2_ref[...], ls_ref[...], lb_ref[...])
    h_ref[...] = h
    ps_ref[...] = h @ ws_ref[...]
    pd_ref[...] = h @ wd_ref[...]
    r8_ref[...] = jnp.concatenate(
        [recent, jnp.zeros((cur.shape[0], LAT - 3), jnp.float32)], axis=1)


def _node_encoder(cur18, ptype2, type_emb, w1, b1, w2, b2, ls, lb, ws, wd):
    n = cur18.shape[0]
    bn = 2000
    grid = (n // bn,)
    blk_n = lambda c: pl.BlockSpec((bn, c), lambda i: (i, 0))
    full = lambda a: pl.BlockSpec(a.shape, lambda i: (0,) * a.ndim)
    return pl.pallas_call(
        _node_encoder_body,
        grid=grid,
        in_specs=[blk_n(18), blk_n(1), full(type_emb), full(w1), full(b1),
                  full(w2), full(b2), full(ls), full(lb), full(ws), full(wd)],
        out_specs=[blk_n(LAT), blk_n(LAT), blk_n(LAT), blk_n(LAT)],
        out_shape=[jax.ShapeDtypeStruct((n, LAT), jnp.float32)] * 4,
    )(cur18, ptype2, type_emb, w1, b1, w2, b2, ls, lb, ws, wd)


def _edge_encoder_body(rd_ref, w1_ref, b1_ref, w2_ref, b2_ref,
                       ls_ref, lb_ref, e_ref):
    disp = rd_ref[...][:, 0:3] * (1.0 / RADIUS)
    nrm = jnp.sqrt(jnp.sum(disp * disp, axis=1, keepdims=True))
    w1 = w1_ref[...]
    pre = disp @ w1[0:3] + nrm @ w1[3:4] + b1_ref[...]
    hid = jnp.maximum(pre, 0.0)
    e_ref[...] = _ln(hid @ w2_ref[...] + b2_ref[...], ls_ref[...],
                     lb_ref[...])


def _edge_encoder(rd8, w1, b1, w2, b2, ls, lb):
    e = rd8.shape[0]
    be = 4096
    grid = (e // be,)
    blk = lambda c: pl.BlockSpec((be, c), lambda i: (i, 0))
    full = lambda a: pl.BlockSpec(a.shape, lambda i: (0,) * a.ndim)
    return pl.pallas_call(
        _edge_encoder_body,
        grid=grid,
        in_specs=[blk(LAT), full(w1), full(b1), full(w2), full(b2),
                  full(ls), full(lb)],
        out_specs=blk(LAT),
        out_shape=jax.ShapeDtypeStruct((e, LAT), jnp.float32),
    )(rd8, w1, b1, w2, b2, ls, lb)


def _edge_mp_body(e_ref, pre_ref, we_ref, b1_ref, w2_ref, b2_ref,
                  out_ref):
    ev = e_ref[...]
    t = jnp.maximum(ev @ we_ref[...] + pre_ref[...] + b1_ref[...], 0.0)
    out_ref[...] = ev + t @ w2_ref[...] + b2_ref[...]


def _edge_mp(e, pre, we, b1, w2, b2):
    ne = e.shape[0]
    be = 4096
    grid = (ne // be,)
    blk = pl.BlockSpec((be, LAT), lambda i: (i, 0))
    full = lambda a: pl.BlockSpec(a.shape, lambda i: (0,) * a.ndim)
    return pl.pallas_call(
        _edge_mp_body,
        grid=grid,
        in_specs=[blk, blk, full(we), full(b1), full(w2), full(b2)],
        out_specs=blk,
        out_shape=jax.ShapeDtypeStruct((ne, LAT), jnp.float32),
    )(e, pre, we, b1, w2, b2)


def _node_mp_body(h_ref, agg_ref, wh_ref, wa_ref, b1_ref, w2_ref,
                  b2_ref, ws_ref, wd_ref, hn_ref, ps_ref, pd_ref):
    h = h_ref[...]
    agg = agg_ref[0] + agg_ref[1]
    t = jnp.maximum(h @ wh_ref[...] + agg @ wa_ref[...] + b1_ref[...], 0.0)
    hn = h + t @ w2_ref[...] + b2_ref[...]
    hn_ref[...] = hn
    ps_ref[...] = hn @ ws_ref[...]
    pd_ref[...] = hn @ wd_ref[...]


def _node_mp(h, agg2, wh, wa, b1, w2, b2, ws, wd):
    n = h.shape[0]
    bn = 2000
    grid = (n // bn,)
    blk = pl.BlockSpec((bn, LAT), lambda i: (i, 0))
    blk2 = pl.BlockSpec((2, bn, LAT), lambda i: (0, i, 0))
    full = lambda a: pl.BlockSpec(a.shape, lambda i: (0,) * a.ndim)
    return pl.pallas_call(
        _node_mp_body,
        grid=grid,
        in_specs=[blk, blk2, full(wh), full(wa), full(b1), full(w2),
                  full(b2), full(ws), full(wd)],
        out_specs=[blk, blk, blk],
        out_shape=[jax.ShapeDtypeStruct((n, LAT), jnp.float32)] * 3,
    )(h, agg2, wh, wa, b1, w2, b2, ws, wd)


def _node_last_body(h_ref, agg_ref, wh_ref, wa_ref, b1_ref,
                    w2_ref, b2_ref, dw1_ref, db1_ref, dw2_ref, db2_ref,
                    cur_ref, gt_ref, kin_ref, nxt_ref):
    h = h_ref[...]
    agg = agg_ref[0] + agg_ref[1]
    t = jnp.maximum(h @ wh_ref[...] + agg @ wa_ref[...] + b1_ref[...], 0.0)
    hn = h + t @ w2_ref[...] + b2_ref[...]
    dh = jnp.maximum(hn @ dw1_ref[...] + db1_ref[...], 0.0)
    acc = (dh @ dw2_ref[...] + db2_ref[...]) * 1e-3
    cur = cur_ref[...]
    recent = cur[:, 15:18]
    vlast = recent - cur[:, 12:15]
    nxt = recent + vlast + acc[:, 0:3]
    kin = kin_ref[...] != 0
    nxt_ref[...] = jnp.where(kin, gt_ref[...], nxt)


def _node_last(h, agg2, wh, wa, b1, w2, b2, dw1, db1, dw2p, db2p,
               cur18, gt_step, kin2):
    n = h.shape[0]
    bn = 2000
    grid = (n // bn,)
    blk = lambda c: pl.BlockSpec((bn, c), lambda i: (i, 0))
    blk2 = pl.BlockSpec((2, bn, LAT), lambda i: (0, i, 0))
    full = lambda a: pl.BlockSpec(a.shape, lambda i: (0,) * a.ndim)
    return pl.pallas_call(
        _node_last_body,
        grid=grid,
        in_specs=[blk(LAT), blk2, full(wh), full(wa), full(b1),
                  full(w2), full(b2), full(dw1), full(db1), full(dw2p),
                  full(db2p), blk(18), blk(D), blk(1)],
        out_specs=blk(D),
        out_shape=jax.ShapeDtypeStruct((n, D), jnp.float32),
    )(h, agg2, wh, wa, b1, w2, b2, dw1, db1, dw2p, db2p, cur18,
      gt_step, kin2)


def _loss_body(p_ref, g_ref, nk_ref, loss_ref):
    d = p_ref[...] - g_ref[...]
    sq = jnp.sum(d * d, axis=2)
    nk = nk_ref[...]
    num = jnp.sum(sq * nk)
    loss_ref[...] = (num / jnp.sum(nk)).reshape(1, 1)


def _loss(preds, gt_p, nonkin):
    n = nonkin.shape[1]
    full = lambda a: pl.BlockSpec(a.shape, lambda: (0,) * a.ndim)
    out = pl.pallas_call(
        _loss_body,
        in_specs=[full(preds), full(gt_p), full(nonkin)],
        out_specs=pl.BlockSpec((1, 1), lambda: (0, 0)),
        out_shape=jax.ShapeDtypeStruct((1, 1), jnp.float32),
    )(preds, gt_p, nonkin)
    return out[0, 0]


def _sc_gather_sum(ps, pd, src2d, dst2d, sign):
    n_chunks, ch = src2d.shape
    n_edges = n_chunks * ch
    w = ps.shape[1]
    dt = ps.dtype
    npt = n_chunks // _NW
    nb = 3
    mesh = plsc.VectorSubcoreMesh(core_axis_name="c", subcore_axis_name="s")

    @functools.partial(
        pl.kernel,
        out_type=jax.ShapeDtypeStruct((n_edges, w), dt),
        mesh=mesh,
        scratch_types=[
            pltpu.VMEM((npt, _CH), jnp.int32),
            pltpu.VMEM((npt, _CH), jnp.int32),
            pltpu.VMEM((nb, _CH, w), dt),
            pltpu.VMEM((nb, _CH, w), dt),
            pltpu.SemaphoreType.DMA((nb,)),
            pltpu.SemaphoreType.DMA((nb,)),
            pltpu.SemaphoreType.DMA((nb,)),
        ],
    )
    def k(ps_hbm, pd_hbm, src_hbm, dst_hbm, out_hbm,
          idxs, idxd, bufa, bufb, sema, semb, semw):
        wid = lax.axis_index("s") * _NC + lax.axis_index("c")
        row0 = wid * npt
        pltpu.sync_copy(src_hbm.at[pl.ds(row0, npt)], idxs)
        pltpu.sync_copy(dst_hbm.at[pl.ds(row0, npt)], idxd)
        for j in range(nb - 1):
            pltpu.async_copy(ps_hbm.at[idxs.at[j]], bufa.at[j], sema.at[j])
            pltpu.async_copy(pd_hbm.at[idxd.at[j]], bufb.at[j], semb.at[j])

        def body(g, _):
            slot = lax.rem(g, nb)
            pltpu.make_async_copy(ps_hbm.at[idxs.at[g]], bufa.at[slot],
                                  sema.at[slot]).wait()
            pltpu.make_async_copy(pd_hbm.at[idxd.at[g]], bufb.at[slot],
                                  semb.at[slot]).wait()
            i = g + nb - 1

            @pl.when(i < npt)
            def _():
                islot = lax.rem(i, nb)

                @pl.when(i >= nb)
                def _():
                    off2 = (row0 + i - nb) * _CH
                    pltpu.make_async_copy(
                        bufa.at[islot], out_hbm.at[pl.ds(off2, _CH)],
                        semw.at[islot]).wait()

                pltpu.async_copy(ps_hbm.at[idxs.at[i]], bufa.at[islot],
                                 sema.at[islot])
                pltpu.async_copy(pd_hbm.at[idxd.at[i]], bufb.at[islot],
                                 semb.at[islot])

            def vrow(r, _):
                for c in range(w // 16):
                    sl = pl.ds(c * 16, 16)
                    if sign >= 0:
                        bufa[slot, r, sl] = bufa[slot, r, sl] + bufb[slot, r, sl]
                    else:
                        bufa[slot, r, sl] = bufa[slot, r, sl] - bufb[slot, r, sl]
                return ()

            lax.fori_loop(0, ch, vrow, ())
            off = (row0 + g) * _CH
            pltpu.async_copy(bufa.at[slot], out_hbm.at[pl.ds(off, _CH)],
                             semw.at[slot])
            return ()

        lax.fori_loop(0, npt, body, ())
        for j in range(nb):
            g = npt - nb + j
            off = (row0 + g) * _CH
            pltpu.make_async_copy(bufa.at[g % nb],
                                  out_hbm.at[pl.ds(off, _CH)],
                                  semw.at[g % nb]).wait()

    return k(ps, pd, src2d, dst2d)


def _sc_scatter(e, dst2d, zeros_n):
    n_edges, lat = e.shape
    ch = dst2d.shape[1]
    n = zeros_n.shape[0]
    rows_t = n // _NS
    n_chunks_s = (n_edges // ch) // _NW
    mesh = plsc.VectorSubcoreMesh(core_axis_name="c", subcore_axis_name="s")

    @functools.partial(
        pl.kernel,
        out_type=jax.ShapeDtypeStruct((_NC, n, lat), e.dtype),
        mesh=mesh,
        scratch_types=[
            pltpu.VMEM((n_chunks_s, ch), jnp.int32),
            pltpu.VMEM((2, ch, lat), e.dtype),
            pltpu.VMEM_SHARED((n, lat), e.dtype),
            pltpu.SemaphoreType.DMA,
            pltpu.SemaphoreType.DMA,
        ],
    )
    def k(e_hbm, dst_hbm, z_hbm, out_hbm, idxd, buf, agg, seml, sems):
        cid = lax.axis_index("c")
        sid = lax.axis_index("s")
        row0 = (cid * _NS + sid) * n_chunks_s
        pltpu.sync_copy(dst_hbm.at[pl.ds(row0, n_chunks_s)], idxd)
        r0 = sid * rows_t
        pltpu.sync_copy(z_hbm.at[pl.ds(r0, rows_t)],
                        agg.at[pl.ds(r0, rows_t)])
        plsc.subcore_barrier()
        pltpu.async_copy(e_hbm.at[pl.ds(row0 * ch, ch)], buf.at[0], seml)

        def body(g, _):
            cur_s = lax.rem(g, 2)
            nxt_s = 1 - cur_s
            pltpu.make_async_copy(e_hbm.at[pl.ds((row0 + g) * ch, ch)],
                                  buf.at[cur_s], seml).wait()

            @pl.when(g >= 1)
            def _():
                pltpu.make_async_copy(buf.at[nxt_s], agg.at[idxd.at[g - 1]],
                                      sems).wait()

            @pl.when(g + 1 < n_chunks_s)
            def _():
                pltpu.async_copy(
                    e_hbm.at[pl.ds((row0 + g + 1) * ch, ch)],
                    buf.at[nxt_s], seml)

            pltpu.async_copy(buf.at[cur_s], agg.at[idxd.at[g]], sems,
                             add=True)
            return ()

        lax.fori_loop(0, n_chunks_s, body, ())
        last = n_chunks_s - 1
        pltpu.make_async_copy(buf.at[last % 2], agg.at[idxd.at[last]],
                              sems).wait()
        plsc.subcore_barrier()
        pltpu.sync_copy(agg.at[pl.ds(r0, rows_t)],
                        out_hbm.at[cid, pl.ds(r0, rows_t)])

    return k(e, dst2d, zeros_n)


def kernel(position, step_context, type_emb, enW1, enb1, enW2, enb2, enls,
           enlb, eeW1, eeb1, eeW2, eeb2, eels, eelb, mpeW1, mpeb1, mpeW2,
           mpeb2, mpnW1, mpnb1, mpnW2, mpnb2, decW1, decb1, decW2, decb2,
           edge_index, particle_type):
    n = position.shape[0]
    n_edges = edge_index.shape[1]
    quantum = _NW * _CH
    e_pad = -(-n_edges // quantum) * quantum
    pad = e_pad - n_edges
    src_p = jnp.concatenate([edge_index[0], jnp.zeros((pad,), jnp.int32)])
    dst_g = jnp.concatenate([edge_index[1], jnp.zeros((pad,), jnp.int32)])
    dst_s = jnp.concatenate([edge_index[1], jnp.full((pad,), n, jnp.int32)])
    src2d = src_p.reshape(e_pad // _CH, _CH)
    dst2d_g = dst_g.reshape(e_pad // _CH, _CH)
    dst2d_s = dst_s.reshape(e_pad // 64, 64)
    rows_t = -(-(n + 1) // (_NS * 16)) * 16
    n_pad = rows_t * _NS
    kin = particle_type == 3
    kin2 = kin.astype(jnp.int32)[:, None]
    ptype2 = particle_type[:, None]
    initial = position[:, :INPUT_SEQ]
    gt = position[:, INPUT_SEQ:INPUT_SEQ + STEPS]
    zeros_n = jnp.zeros((n_pad, LAT), jnp.float32)

    we = [mpeW1[m][0:LAT] for m in range(MP)]
    ws = [mpeW1[m][LAT:2 * LAT] for m in range(MP)]
    wd = [mpeW1[m][2 * LAT:3 * LAT] for m in range(MP)]
    wh = [mpnW1[m][0:LAT] for m in range(MP)]
    wa = [mpnW1[m][LAT:2 * LAT] for m in range(MP)]
    row = lambda v: v.reshape(1, LAT)
    meb1 = [row(mpeb1[m]) for m in range(MP)]
    meb2 = [row(mpeb2[m]) for m in range(MP)]
    mew2 = [mpeW2[m] for m in range(MP)]
    mnb1 = [row(mpnb1[m]) for m in range(MP)]
    mnb2 = [row(mpnb2[m]) for m in range(MP)]
    dw2p = jnp.zeros((LAT, 8), jnp.float32).at[:, 0:D].set(decW2)
    db2p = jnp.zeros((1, 8), jnp.float32).at[0, 0:D].set(decb2)

    cur = initial
    preds = []
    for step in range(STEPS):
        cur18 = cur.reshape(n, INPUT_SEQ * D)
        h, psn, pdn, r8 = _node_encoder(cur18, ptype2, type_emb, enW1,
                                        row(enb1), enW2, row(enb2),
                                        row(enls), row(enlb), ws[0], wd[0])
        rdiff = _sc_gather_sum(r8, r8, src2d, dst2d_g, -1.0)
        e = _edge_encoder(rdiff, eeW1, row(eeb1), eeW2, row(eeb2),
                          row(eels), row(eelb))
        for m in range(MP):
            pre = _sc_gather_sum(psn, pdn, src2d, dst2d_g, 1.0)
            e = _edge_mp(e, pre, we[m], meb1[m], mew2[m], meb2[m])
            agg2 = _sc_scatter(e, dst2d_s, zeros_n)
            if m + 1 < MP:
                h, psn, pdn = _node_mp(h, agg2, wh[m], wa[m], mnb1[m],
                                       mpnW2[m], mnb2[m], ws[m + 1],
                                       wd[m + 1])
            else:
                nxt = _node_last(h, agg2, wh[m], wa[m], mnb1[m], mpnW2[m],
                                 mnb2[m], decW1, row(decb1), dw2p, db2p,
                                 cur18, gt[:, step], kin2)
        preds.append(nxt)
        cur = jnp.concatenate([cur[:, 1:], nxt[:, None, :]], axis=1)

    predictions = jnp.stack(preds)
    gt_p = jnp.transpose(gt, (1, 0, 2))
    nonkin = (~kin).astype(jnp.float32)[None, :]
    loss = _loss(predictions, gt_p, nonkin)
    return (loss, predictions, gt_p)


# speedup vs baseline: 1.1298x; 1.0707x over previous
import functools

import jax
import jax.numpy as jnp
from jax import lax
from jax.experimental import pallas as pl
from jax.experimental.pallas import tpu as pltpu
from jax.experimental.pallas import tpu_sc as plsc

INPUT_SEQ = 6
STEPS = 2
D = 3
RADIUS = 0.015
MP = 10
LAT = 128

_NC = 2
_NS = 16
_NW = _NC * _NS
_CH = 128


def _ln(x, s, b):
    m = jnp.mean(x, axis=-1, keepdims=True)
    xc = x - m
    v = jnp.mean(xc * xc, axis=-1, keepdims=True)
    return xc * lax.rsqrt(v + 1e-5) * s + b


def _node_encoder_body(cur_ref, pt_ref, temb_ref, w1_ref, b1_ref, w2_ref,
                       b2_ref, ls_ref, lb_ref, ws_ref, wd_ref,
                       h_ref, ps_ref, pd_ref, r8_ref):
    cur = cur_ref[...]
    vel = cur[:, 3:18] - cur[:, 0:15]
    recent = cur[:, 15:18]
    bdist = jnp.clip(jnp.minimum(recent, 1.0 - recent) / RADIUS, -1.0, 1.0)
    w1 = w1_ref[...]
    pt = pt_ref[...]
    onehot = (pt == lax.broadcasted_iota(jnp.int32, (pt.shape[0], 9), 1)
              ).astype(jnp.float32)
    emb_proj = temb_ref[...] @ w1[21:37]
    ctx = (0.5 * w1[37] + 0.00025 * w1[38] + 0.2 * w1[39])[None, :]
    pre = (vel @ w1[0:15] + recent @ w1[15:18] + bdist @ w1[18:21]
           + onehot @ emb_proj + ctx + b1_ref[...])
    hid = jnp.maximum(pre, 0.0)
    h = _ln(hid @ w2_ref[...] + b2_ref[...], ls_ref[...], lb_ref[...])
    h_ref[...] = h
    ps_ref[...] = h @ ws_ref[...]
    pd_ref[...] = h @ wd_ref[...]
    r8_ref[...] = jnp.concatenate(
        [recent, jnp.zeros((cur.shape[0], LAT - 3), jnp.float32)], axis=1)


def _node_encoder(cur18, ptype2, type_emb, w1, b1, w2, b2, ls, lb, ws, wd):
    n = cur18.shape[0]
    bn = 2000
    grid = (n // bn,)
    blk_n = lambda c: pl.BlockSpec((bn, c), lambda i: (i, 0))
    full = lambda a: pl.BlockSpec(a.shape, lambda i: (0,) * a.ndim)
    return pl.pallas_call(
        _node_encoder_body,
        grid=grid,
        in_specs=[blk_n(18), blk_n(1), full(type_emb), full(w1), full(b1),
                  full(w2), full(b2), full(ls), full(lb), full(ws), full(wd)],
        out_specs=[blk_n(LAT), blk_n(LAT), blk_n(LAT), blk_n(LAT)],
        out_shape=[jax.ShapeDtypeStruct((n, LAT), jnp.float32)] * 4,
    )(cur18, ptype2, type_emb, w1, b1, w2, b2, ls, lb, ws, wd)


def _edge_encoder_body(rd_ref, w1_ref, b1_ref, w2_ref, b2_ref,
                       ls_ref, lb_ref, e_ref):
    disp = rd_ref[...][:, 0:3] * (1.0 / RADIUS)
    nrm = jnp.sqrt(jnp.sum(disp * disp, axis=1, keepdims=True))
    w1 = w1_ref[...]
    pre = disp @ w1[0:3] + nrm @ w1[3:4] + b1_ref[...]
    hid = jnp.maximum(pre, 0.0)
    e_ref[...] = _ln(hid @ w2_ref[...] + b2_ref[...], ls_ref[...],
                     lb_ref[...])


def _edge_encoder(rd8, w1, b1, w2, b2, ls, lb):
    e = rd8.shape[0]
    be = 4096
    grid = (e // be,)
    blk = lambda c: pl.BlockSpec((be, c), lambda i: (i, 0))
    full = lambda a: pl.BlockSpec(a.shape, lambda i: (0,) * a.ndim)
    return pl.pallas_call(
        _edge_encoder_body,
        grid=grid,
        in_specs=[blk(LAT), full(w1), full(b1), full(w2), full(b2),
                  full(ls), full(lb)],
        out_specs=blk(LAT),
        out_shape=jax.ShapeDtypeStruct((e, LAT), jnp.float32),
    )(rd8, w1, b1, w2, b2, ls, lb)


def _edge_mp_body(e_ref, pre_ref, we_ref, b1_ref, w2_ref, b2_ref,
                  out_ref):
    ev = e_ref[...]
    t = jnp.maximum(ev @ we_ref[...] + pre_ref[...] + b1_ref[...], 0.0)
    out_ref[...] = ev + t @ w2_ref[...] + b2_ref[...]


def _edge_mp(e, pre, we, b1, w2, b2):
    ne = e.shape[0]
    be = 4096
    grid = (ne // be,)
    blk = pl.BlockSpec((be, LAT), lambda i: (i, 0))
    full = lambda a: pl.BlockSpec(a.shape, lambda i: (0,) * a.ndim)
    return pl.pallas_call(
        _edge_mp_body,
        grid=grid,
        in_specs=[blk, blk, full(we), full(b1), full(w2), full(b2)],
        out_specs=blk,
        out_shape=jax.ShapeDtypeStruct((ne, LAT), jnp.float32),
    )(e, pre, we, b1, w2, b2)


def _node_mp_body(h_ref, agg_ref, wh_ref, wa_ref, b1_ref, w2_ref,
                  b2_ref, ws_ref, wd_ref, hn_ref, ps_ref, pd_ref):
    h = h_ref[...]
    agg = agg_ref[0] + agg_ref[1]
    t = jnp.maximum(h @ wh_ref[...] + agg @ wa_ref[...] + b1_ref[...], 0.0)
    hn = h + t @ w2_ref[...] + b2_ref[...]
    hn_ref[...] = hn
    ps_ref[...] = hn @ ws_ref[...]
    pd_ref[...] = hn @ wd_ref[...]


def _node_mp(h, agg2, wh, wa, b1, w2, b2, ws, wd):
    n = h.shape[0]
    bn = 2000
    grid = (n // bn,)
    blk = pl.BlockSpec((bn, LAT), lambda i: (i, 0))
    blk2 = pl.BlockSpec((2, bn, LAT), lambda i: (0, i, 0))
    full = lambda a: pl.BlockSpec(a.shape, lambda i: (0,) * a.ndim)
    return pl.pallas_call(
        _node_mp_body,
        grid=grid,
        in_specs=[blk, blk2, full(wh), full(wa), full(b1), full(w2),
                  full(b2), full(ws), full(wd)],
        out_specs=[blk, blk, blk],
        out_shape=[jax.ShapeDtypeStruct((n, LAT), jnp.float32)] * 3,
    )(h, agg2, wh, wa, b1, w2, b2, ws, wd)


def _node_last_body(h_ref, agg_ref, wh_ref, wa_ref, b1_ref,
                    w2_ref, b2_ref, dw1_ref, db1_ref, dw2_ref, db2_ref,
                    cur_ref, gt_ref, kin_ref, nxt_ref):
    h = h_ref[...]
    agg = agg_ref[0] + agg_ref[1]
    t = jnp.maximum(h @ wh_ref[...] + agg @ wa_ref[...] + b1_ref[...], 0.0)
    hn = h + t @ w2_ref[...] + b2_ref[...]
    dh = jnp.maximum(hn @ dw1_ref[...] + db1_ref[...], 0.0)
    acc = (dh @ dw2_ref[...] + db2_ref[...]) * 1e-3
    cur = cur_ref[...]
    recent = cur[:, 15:18]
    vlast = recent - cur[:, 12:15]
    nxt = recent + vlast + acc[:, 0:3]
    kin = kin_ref[...] != 0
    nxt_ref[...] = jnp.where(kin, gt_ref[...], nxt)


def _node_last(h, agg2, wh, wa, b1, w2, b2, dw1, db1, dw2p, db2p,
               cur18, gt_step, kin2):
    n = h.shape[0]
    bn = 2000
    grid = (n // bn,)
    blk = lambda c: pl.BlockSpec((bn, c), lambda i: (i, 0))
    blk2 = pl.BlockSpec((2, bn, LAT), lambda i: (0, i, 0))
    full = lambda a: pl.BlockSpec(a.shape, lambda i: (0,) * a.ndim)
    return pl.pallas_call(
        _node_last_body,
        grid=grid,
        in_specs=[blk(LAT), blk2, full(wh), full(wa), full(b1),
                  full(w2), full(b2), full(dw1), full(db1), full(dw2p),
                  full(db2p), blk(18), blk(D), blk(1)],
        out_specs=blk(D),
        out_shape=jax.ShapeDtypeStruct((n, D), jnp.float32),
    )(h, agg2, wh, wa, b1, w2, b2, dw1, db1, dw2p, db2p, cur18,
      gt_step, kin2)


def _loss_body(p_ref, g_ref, nk_ref, loss_ref):
    d = p_ref[...] - g_ref[...]
    sq = jnp.sum(d * d, axis=2)
    nk = nk_ref[...]
    num = jnp.sum(sq * nk)
    loss_ref[...] = (num / jnp.sum(nk)).reshape(1, 1)


def _loss(preds, gt_p, nonkin):
    n = nonkin.shape[1]
    full = lambda a: pl.BlockSpec(a.shape, lambda: (0,) * a.ndim)
    out = pl.pallas_call(
        _loss_body,
        in_specs=[full(preds), full(gt_p), full(nonkin)],
        out_specs=pl.BlockSpec((1, 1), lambda: (0, 0)),
        out_shape=jax.ShapeDtypeStruct((1, 1), jnp.float32),
    )(preds, gt_p, nonkin)
    return out[0, 0]


def _sc_gather_sum(ps, pd, src2d, dst2d, sign):
    n_chunks, ch = src2d.shape
    n_edges = n_chunks * ch
    w = ps.shape[1]
    dt = ps.dtype
    npt = n_chunks // _NW
    nb = 3
    mesh = plsc.VectorSubcoreMesh(core_axis_name="c", subcore_axis_name="s")

    @functools.partial(
        pl.kernel,
        out_type=jax.ShapeDtypeStruct((n_edges, w), dt),
        mesh=mesh,
        scratch_types=[
            pltpu.VMEM((npt, _CH), jnp.int32),
            pltpu.VMEM((npt, _CH), jnp.int32),
            pltpu.VMEM((nb, _CH, w), dt),
            pltpu.VMEM((nb, _CH, w), dt),
            pltpu.SemaphoreType.DMA((nb,)),
            pltpu.SemaphoreType.DMA((nb,)),
            pltpu.SemaphoreType.DMA((nb,)),
        ],
    )
    def k(ps_hbm, pd_hbm, src_hbm, dst_hbm, out_hbm,
          idxs, idxd, bufa, bufb, sema, semb, semw):
        wid = lax.axis_index("s") * _NC + lax.axis_index("c")
        row0 = wid * npt
        pltpu.sync_copy(src_hbm.at[pl.ds(row0, npt)], idxs)
        pltpu.sync_copy(dst_hbm.at[pl.ds(row0, npt)], idxd)
        for j in range(nb - 1):
            pltpu.async_copy(ps_hbm.at[idxs.at[j]], bufa.at[j], sema.at[j])
            pltpu.async_copy(pd_hbm.at[idxd.at[j]], bufb.at[j], semb.at[j])

        def body(g, _):
            slot = lax.rem(g, nb)
            pltpu.make_async_copy(ps_hbm.at[idxs.at[g]], bufa.at[slot],
                                  sema.at[slot]).wait()
            pltpu.make_async_copy(pd_hbm.at[idxd.at[g]], bufb.at[slot],
                                  semb.at[slot]).wait()
            i = g + nb - 1

            @pl.when(i < npt)
            def _():
                islot = lax.rem(i, nb)

                @pl.when(i >= nb)
                def _():
                    off2 = (row0 + i - nb) * _CH
                    pltpu.make_async_copy(
                        bufa.at[islot], out_hbm.at[pl.ds(off2, _CH)],
                        semw.at[islot]).wait()

                pltpu.async_copy(ps_hbm.at[idxs.at[i]], bufa.at[islot],
                                 sema.at[islot])
                pltpu.async_copy(pd_hbm.at[idxd.at[i]], bufb.at[islot],
                                 semb.at[islot])

            def vrow(r, _):
                for c in range(w // 16):
                    sl = pl.ds(c * 16, 16)
                    if sign >= 0:
                        bufa[slot, r, sl] = bufa[slot, r, sl] + bufb[slot, r, sl]
                    else:
                        bufa[slot, r, sl] = bufa[slot, r, sl] - bufb[slot, r, sl]
                return ()

            lax.fori_loop(0, ch, vrow, ())
            off = (row0 + g) * _CH
            pltpu.async_copy(bufa.at[slot], out_hbm.at[pl.ds(off, _CH)],
                             semw.at[slot])
            return ()

        lax.fori_loop(0, npt, body, ())
        for j in range(nb):
            g = npt - nb + j
            off = (row0 + g) * _CH
            pltpu.make_async_copy(bufa.at[g % nb],
                                  out_hbm.at[pl.ds(off, _CH)],
                                  semw.at[g % nb]).wait()

    return k(ps, pd, src2d, dst2d)


def _sc_scatter(e, dst2d, zeros_n):
    n_edges, lat = e.shape
    ch = dst2d.shape[1]
    n = zeros_n.shape[0]
    rows_t = n // _NS
    n_chunks_s = (n_edges // ch) // _NW
    mesh = plsc.VectorSubcoreMesh(core_axis_name="c", subcore_axis_name="s")

    @functools.partial(
        pl.kernel,
        out_type=jax.ShapeDtypeStruct((_NC, n, lat), e.dtype),
        mesh=mesh,
        scratch_types=[
            pltpu.VMEM((n_chunks_s, ch), jnp.int32),
            pltpu.VMEM((4, ch, lat), e.dtype),
            pltpu.VMEM_SHARED((n, lat), e.dtype),
            pltpu.SemaphoreType.DMA((4,)),
            pltpu.SemaphoreType.DMA((4,)),
        ],
    )
    def k(e_hbm, dst_hbm, z_hbm, out_hbm, idxd, buf, agg, seml, sems):
        nb = 4
        cid = lax.axis_index("c")
        sid = lax.axis_index("s")
        row0 = (cid * _NS + sid) * n_chunks_s
        pltpu.sync_copy(dst_hbm.at[pl.ds(row0, n_chunks_s)], idxd)
        r0 = sid * rows_t
        pltpu.sync_copy(z_hbm.at[pl.ds(r0, rows_t)],
                        agg.at[pl.ds(r0, rows_t)])
        plsc.subcore_barrier()
        for j in range(nb - 1):
            pltpu.async_copy(e_hbm.at[pl.ds((row0 + j) * ch, ch)],
                             buf.at[j], seml.at[j])

        def body(g, _):
            slot = lax.rem(g, nb)
            pltpu.make_async_copy(e_hbm.at[pl.ds((row0 + g) * ch, ch)],
                                  buf.at[slot], seml.at[slot]).wait()
            i = g + nb - 1

            @pl.when(i < n_chunks_s)
            def _():
                islot = lax.rem(i, nb)

                @pl.when(i >= nb)
                def _():
                    pltpu.make_async_copy(buf.at[islot],
                                          agg.at[idxd.at[i - nb]],
                                          sems.at[islot]).wait()

                pltpu.async_copy(e_hbm.at[pl.ds((row0 + i) * ch, ch)],
                                 buf.at[islot], seml.at[islot])

            pltpu.async_copy(buf.at[slot], agg.at[idxd.at[g]], sems.at[slot],
                             add=True)
            return ()

        lax.fori_loop(0, n_chunks_s, body, ())
        for j in range(nb):
            g = n_chunks_s - nb + j
            pltpu.make_async_copy(buf.at[g % nb], agg.at[idxd.at[g]],
                                  sems.at[g % nb]).wait()
        plsc.subcore_barrier()
        pltpu.sync_copy(agg.at[pl.ds(r0, rows_t)],
                        out_hbm.at[cid, pl.ds(r0, rows_t)])

    return k(e, dst2d, zeros_n)


def kernel(position, step_context, type_emb, enW1, enb1, enW2, enb2, enls,
           enlb, eeW1, eeb1, eeW2, eeb2, eels, eelb, mpeW1, mpeb1, mpeW2,
           mpeb2, mpnW1, mpnb1, mpnW2, mpnb2, decW1, decb1, decW2, decb2,
           edge_index, particle_type):
    n = position.shape[0]
    n_edges = edge_index.shape[1]
    quantum = _NW * _CH
    e_pad = -(-n_edges // quantum) * quantum
    pad = e_pad - n_edges
    src_p = jnp.concatenate([edge_index[0], jnp.zeros((pad,), jnp.int32)])
    dst_g = jnp.concatenate([edge_index[1], jnp.zeros((pad,), jnp.int32)])
    dst_s = jnp.concatenate([edge_index[1], jnp.full((pad,), n, jnp.int32)])
    src2d = src_p.reshape(e_pad // _CH, _CH)
    dst2d_g = dst_g.reshape(e_pad // _CH, _CH)
    dst2d_s = dst_s.reshape(e_pad // 64, 64)
    rows_t = -(-(n + 1) // (_NS * 16)) * 16
    n_pad = rows_t * _NS
    kin = particle_type == 3
    kin2 = kin.astype(jnp.int32)[:, None]
    ptype2 = particle_type[:, None]
    initial = position[:, :INPUT_SEQ]
    gt = position[:, INPUT_SEQ:INPUT_SEQ + STEPS]
    zeros_n = jnp.zeros((n_pad, LAT), jnp.float32)

    we = [mpeW1[m][0:LAT] for m in range(MP)]
    ws = [mpeW1[m][LAT:2 * LAT] for m in range(MP)]
    wd = [mpeW1[m][2 * LAT:3 * LAT] for m in range(MP)]
    wh = [mpnW1[m][0:LAT] for m in range(MP)]
    wa = [mpnW1[m][LAT:2 * LAT] for m in range(MP)]
    row = lambda v: v.reshape(1, LAT)
    meb1 = [row(mpeb1[m]) for m in range(MP)]
    meb2 = [row(mpeb2[m]) for m in range(MP)]
    mew2 = [mpeW2[m] for m in range(MP)]
    mnb1 = [row(mpnb1[m]) for m in range(MP)]
    mnb2 = [row(mpnb2[m]) for m in range(MP)]
    dw2p = jnp.zeros((LAT, 8), jnp.float32).at[:, 0:D].set(decW2)
    db2p = jnp.zeros((1, 8), jnp.float32).at[0, 0:D].set(decb2)

    cur = initial
    preds = []
    for step in range(STEPS):
        cur18 = cur.reshape(n, INPUT_SEQ * D)
        h, psn, pdn, r8 = _node_encoder(cur18, ptype2, type_emb, enW1,
                                        row(enb1), enW2, row(enb2),
                                        row(enls), row(enlb), ws[0], wd[0])
        rdiff = _sc_gather_sum(r8, r8, src2d, dst2d_g, -1.0)
        e = _edge_encoder(rdiff, eeW1, row(eeb1), eeW2, row(eeb2),
                          row(eels), row(eelb))
        for m in range(MP):
            pre = _sc_gather_sum(psn, pdn, src2d, dst2d_g, 1.0)
            e = _edge_mp(e, pre, we[m], meb1[m], mew2[m], meb2[m])
            agg2 = _sc_scatter(e, dst2d_s, zeros_n)
            if m + 1 < MP:
                h, psn, pdn = _node_mp(h, agg2, wh[m], wa[m], mnb1[m],
                                       mpnW2[m], mnb2[m], ws[m + 1],
                                       wd[m + 1])
            else:
                nxt = _node_last(h, agg2, wh[m], wa[m], mnb1[m], mpnW2[m],
                                 mnb2[m], decW1, row(decb1), dw2p, db2p,
                                 cur18, gt[:, step], kin2)
        preds.append(nxt)
        cur = jnp.concatenate([cur[:, 1:], nxt[:, None, :]], axis=1)

    predictions = jnp.stack(preds)
    gt_p = jnp.transpose(gt, (1, 0, 2))
    nonkin = (~kin).astype(jnp.float32)[None, :]
    loss = _loss(predictions, gt_p, nonkin)
    return (loss, predictions, gt_p)
